# baseline (speedup 1.0000x reference)
import jax
import jax.numpy as jnp
from jax import lax
from jax.experimental import pallas as pl
from jax.experimental.pallas import tpu as pltpu
from jax.experimental.pallas import tpu_sc as plsc

VOCAB = 1000000
MAX_SEQ = 2048
DIM = 64
BATCH = 32

NUM_CORES = 2
CHUNK = 256
NUM_CHUNKS = MAX_SEQ // CHUNK
IDX_ROWS = CHUNK // 128
LANES = 16
VECS = DIM // LANES


def _sc_body(x_hbm, tok_hbm, pos_hbm, out_hbm,
             idx_v, rows2_v, out_v, pos_v, sem):
    b = lax.axis_index("s") * NUM_CORES + lax.axis_index("c")

    def chunk_body(c, _):
        off = pl.multiple_of(c * CHUNK, CHUNK)
        for q in range(IDX_ROWS):
            pltpu.sync_copy(
                x_hbm.at[b, pl.ds(off + q * 128, 128)], idx_v.at[q])

        copies = [
            pltpu.async_copy(
                tok_hbm.at[idx_v.at[q]],
                rows2_v.at[pl.ds(q * 128, 128), :], sem)
            for q in range(IDX_ROWS)
        ]
        pltpu.sync_copy(pos_hbm.at[pl.ds(off, CHUNK), :], pos_v)
        for cp in copies:
            cp.wait()

        def add_row(j, _):
            for v in range(VECS):
                s = pl.ds(v * LANES, LANES)
                out_v[j, s] = rows2_v[j, s] + pos_v[j, s]
            return 0

        lax.fori_loop(0, CHUNK, add_row, 0)
        pltpu.sync_copy(out_v, out_hbm.at[pl.ds(b * MAX_SEQ + off, CHUNK), :])
        return 0

    lax.fori_loop(0, NUM_CHUNKS, chunk_body, 0)


@jax.jit
def kernel(x, token_table, pos_table):
    tok_wide = jnp.pad(token_table, ((0, 0), (0, DIM)))
    mesh = plsc.VectorSubcoreMesh(core_axis_name="c", subcore_axis_name="s")
    out = pl.kernel(
        _sc_body,
        out_type=jax.ShapeDtypeStruct((BATCH * MAX_SEQ, DIM), jnp.float32),
        mesh=mesh,
        scratch_types=[
            pltpu.VMEM((IDX_ROWS, 128), jnp.int32),
            pltpu.VMEM((CHUNK, 2 * DIM), jnp.float32),
            pltpu.VMEM((CHUNK, DIM), jnp.float32),
            pltpu.VMEM((CHUNK, DIM), jnp.float32),
            pltpu.SemaphoreType.DMA,
        ],
    )(x, tok_wide, pos_table)
    return out.reshape(BATCH, MAX_SEQ, DIM)
